# Initial kernel scaffold; baseline (speedup 1.0000x reference)
import jax
import jax.numpy as jnp
from jax.experimental import pallas as pl


def kernel(x, sldj, l0_M, l0_bvec, l0_w1, l0_b1, l0_w2, l0_b2, l0_w3s, l0_w3t, l0_b3s, l0_b3t, l0_scale, l0_ldj_const, l1_M, l1_bvec, l1_w1, l1_b1, l1_w2, l1_b2, l1_w3s, l1_w3t, l1_b3s, l1_b3t, l1_scale, l1_ldj_const, l2_M, l2_bvec, l2_w1, l2_b1, l2_w2, l2_b2, l2_w3s, l2_w3t, l2_b3s, l2_b3t, l2_scale, l2_ldj_const):
    raise NotImplementedError("write your pallas kernel here")


# speedup vs baseline: 1.0863x; 1.0863x over previous
import functools

import jax
import jax.numpy as jnp
from jax.experimental import pallas as pl
from jax.experimental.pallas import tpu as pltpu

_TAPS = tuple((oy, ox) for oy in (-1, 0, 1) for ox in (-1, 0, 1))


def _level_body(x_ref, m_ref, bv_ref, w1_ref, b1_ref, w2_ref, b2_ref,
                w3s_ref, w3t_ref, b3s_ref, b3t_ref, sc_ref,
                xo_ref, ldj_ref, *, H, W, g, ch):
    HW = H * W
    L = g * HW
    num_steps = m_ref.shape[0]

    x = x_ref[0]

    lane = jax.lax.broadcasted_iota(jnp.int32, (1, L), 1)
    xpos = lane % W
    ypos = (lane // W) % H

    masks = []
    for oy, ox in _TAPS:
        m = None
        if oy == -1:
            m = ypos >= 1
        elif oy == 1:
            m = ypos <= H - 2
        if ox == -1:
            mx = xpos >= 1
            m = mx if m is None else jnp.logical_and(m, mx)
        elif ox == 1:
            mx = xpos <= W - 2
            m = mx if m is None else jnp.logical_and(m, mx)
        masks.append(m)

    def tap_shift(a, k):
        oy, ox = _TAPS[k]
        off = oy * W + ox
        out = a if off == 0 else pltpu.roll(a, (-off) % L, axis=1)
        return out if masks[k] is None else jnp.where(masks[k], out, 0.0)

    ldj_lanes = jnp.zeros((1, L), jnp.float32)
    for s in range(num_steps):
        x = jnp.dot(m_ref[s], x, preferred_element_type=jnp.float32) + bv_ref[s]

        pre = None
        for k in range(9):
            t = jnp.dot(w1_ref[s, k], tap_shift(x, k),
                        preferred_element_type=jnp.float32)
            pre = t if pre is None else pre + t
        h = jnp.maximum(pre + b1_ref[s], 0.0)
        h = jnp.maximum(jnp.dot(w2_ref[s], h, preferred_element_type=jnp.float32)
                        + b2_ref[s], 0.0)
        s_raw = None
        t_val = None
        for k in range(9):
            ds = tap_shift(jnp.dot(w3s_ref[s, k], h,
                                   preferred_element_type=jnp.float32)[:ch], k)
            dt = tap_shift(jnp.dot(w3t_ref[s, k], h,
                                   preferred_element_type=jnp.float32)[:ch], k)
            s_raw = ds if s_raw is None else s_raw + ds
            t_val = dt if t_val is None else t_val + dt

        s_val = sc_ref[s] * jnp.tanh(s_raw + b3s_ref[s])
        t_val = t_val + b3t_ref[s]
        xa = (x[:ch] + t_val) * jnp.exp(s_val)
        x = jnp.concatenate([xa, x[ch:]], axis=0)
        ldj_lanes = ldj_lanes + jnp.sum(s_val, axis=0, keepdims=True)

    xo_ref[0] = x

    if g == 1:
        delta = jnp.sum(ldj_lanes, axis=1, keepdims=True)
    else:
        row = jax.lax.broadcasted_iota(jnp.int32, (g, L), 0)
        seg = row == lane // HW
        delta = jnp.sum(jnp.where(seg, ldj_lanes, 0.0), axis=1,
                        keepdims=True)
    ldj_ref[0] = jnp.broadcast_to(delta, (g, 128))


def _const_spec(a):
    nd = a.ndim
    return pl.BlockSpec(a.shape, lambda b, _n=nd: (0,) * _n)


def _pick_group(N, HW):
    for g in range(1, N + 1):
        if N % g == 0 and (g * HW) % 128 == 0:
            return g
    return N


def _run_level(x, H, W, p):
    N, C, HW = x.shape
    ch = C // 2
    g = _pick_group(N, HW)
    B = N // g
    L = g * HW
    if g == 1:
        xb = x
    else:
        xb = x.reshape(B, g, C, HW).transpose(0, 2, 1, 3).reshape(B, C, L)

    body = functools.partial(_level_body, H=H, W=W, g=g, ch=ch)
    xo, delta = pl.pallas_call(
        body,
        grid=(B,),
        out_shape=(
            jax.ShapeDtypeStruct((B, C, L), jnp.float32),
            jax.ShapeDtypeStruct((B, g, 128), jnp.float32),
        ),
        in_specs=[pl.BlockSpec((1, C, L), lambda b: (b, 0, 0))]
        + [_const_spec(p[k]) for k in
           ("M", "bv", "w1", "b1", "w2", "b2", "w3s", "w3t",
            "b3s", "b3t", "sc")],
        out_specs=(
            pl.BlockSpec((1, C, L), lambda b: (b, 0, 0)),
            pl.BlockSpec((1, g, 128), lambda b: (b, 0, 0)),
        ),
        compiler_params=pltpu.CompilerParams(
            dimension_semantics=("parallel",)),
    )(xb, p["M"], p["bv"], p["w1"], p["b1"], p["w2"], p["b2"],
      p["w3s"], p["w3t"], p["b3s"], p["b3t"], p["sc"])

    if g != 1:
        xo = xo.reshape(B, C, g, HW).transpose(0, 2, 1, 3).reshape(N, C, HW)
    return xo, delta.reshape(N, 128)[:, 0]


def _prep_level(M, bvec, w1, b1, w2, b2, w3s, w3t, b3s, b3t, scale):
    C = M.shape[1]
    ch = C // 2
    return {"M": M, "bv": bvec, "w1": w1, "b1": b1, "w2": w2, "b2": b2,
            "w3s": w3s, "w3t": w3t, "b3s": b3s[:, :ch], "b3t": b3t[:, :ch],
            "sc": scale[:, :ch]}


def _squeeze(x, H, W):
    N, C, _ = x.shape
    x = x.reshape(N, C, H // 2, 2, W // 2, 2)
    x = x.transpose(0, 1, 3, 5, 2, 4)
    return x.reshape(N, 4 * C, (H * W) // 4)


def _unsqueeze(x, H2, W2):
    N, C4, _ = x.shape
    x = x.reshape(N, C4 // 4, 2, 2, H2, W2)
    x = x.transpose(0, 1, 4, 2, 5, 3)
    return x.reshape(N, C4 // 4, H2 * 2 * W2 * 2)


def _forward(x, sldj, levels, i, H, W):
    prep, ldj_const = levels[i]
    x, delta = _run_level(x, H, W, prep)
    sldj = sldj + ldj_const + delta
    if i + 1 < len(levels):
        x = _squeeze(x, H, W)
        c4 = x.shape[1]
        x1, x2 = x[:, : c4 // 2], x[:, c4 // 2:]
        x1, sldj = _forward(x1, sldj, levels, i + 1, H // 2, W // 2)
        x = _unsqueeze(jnp.concatenate([x1, x2], axis=1), H // 2, W // 2)
    return x, sldj


def kernel(x, sldj,
           l0_M, l0_bvec, l0_w1, l0_b1, l0_w2, l0_b2, l0_w3s, l0_w3t,
           l0_b3s, l0_b3t, l0_scale, l0_ldj_const,
           l1_M, l1_bvec, l1_w1, l1_b1, l1_w2, l1_b2, l1_w3s, l1_w3t,
           l1_b3s, l1_b3t, l1_scale, l1_ldj_const,
           l2_M, l2_bvec, l2_w1, l2_b1, l2_w2, l2_b2, l2_w3s, l2_w3t,
           l2_b3s, l2_b3t, l2_scale, l2_ldj_const):
    levels = [
        (_prep_level(l0_M, l0_bvec, l0_w1, l0_b1, l0_w2, l0_b2, l0_w3s,
                     l0_w3t, l0_b3s, l0_b3t, l0_scale), l0_ldj_const),
        (_prep_level(l1_M, l1_bvec, l1_w1, l1_b1, l1_w2, l1_b2, l1_w3s,
                     l1_w3t, l1_b3s, l1_b3t, l1_scale), l1_ldj_const),
        (_prep_level(l2_M, l2_bvec, l2_w1, l2_b1, l2_w2, l2_b2, l2_w3s,
                     l2_w3t, l2_b3s, l2_b3t, l2_scale), l2_ldj_const),
    ]
    n, c, h, w = x.shape
    r = x.reshape(n, c, h * w)
    r, sldj = _forward(r, sldj, levels, 0, h, w)
    return r.reshape(n, c, h, w), sldj


# speedup vs baseline: 1.3116x; 1.2073x over previous
import functools

import jax
import jax.numpy as jnp
from jax.experimental import pallas as pl
from jax.experimental.pallas import tpu as pltpu

_TAPS = tuple((oy, ox) for oy in (-1, 0, 1) for ox in (-1, 0, 1))


def _level_body(x_ref, m_ref, bv_ref, w1_ref, b1_ref, w2_ref, b2_ref,
                w3st_ref, b3s_ref, b3t_ref, sc_ref,
                xo_ref, ldj_ref, *, H, W, g, ch):
    HW = H * W
    L = g * HW
    num_steps = m_ref.shape[0]

    x = x_ref[0]

    lane = jax.lax.broadcasted_iota(jnp.int32, (1, L), 1)
    xpos = lane % W
    ypos = (lane // W) % H

    masks = []
    for oy, ox in _TAPS:
        m = None
        if oy == -1:
            m = ypos >= 1
        elif oy == 1:
            m = ypos <= H - 2
        if ox == -1:
            mx = xpos >= 1
            m = mx if m is None else jnp.logical_and(m, mx)
        elif ox == 1:
            mx = xpos <= W - 2
            m = mx if m is None else jnp.logical_and(m, mx)
        masks.append(m)

    def tap_shift(a, k):
        oy, ox = _TAPS[k]
        off = oy * W + ox
        out = a if off == 0 else pltpu.roll(a, (-off) % L, axis=1)
        return out if masks[k] is None else jnp.where(masks[k], out, 0.0)

    ldj_rows = jnp.zeros((ch, L), jnp.float32)
    for s in range(num_steps):
        x = jnp.dot(m_ref[s], x, preferred_element_type=jnp.float32) + bv_ref[s]

        pre = None
        for k in range(9):
            t = jnp.dot(w1_ref[s, k], tap_shift(x, k),
                        preferred_element_type=jnp.float32)
            pre = t if pre is None else pre + t
        h = jnp.maximum(pre + b1_ref[s], 0.0)
        h = jnp.maximum(jnp.dot(w2_ref[s], h, preferred_element_type=jnp.float32)
                        + b2_ref[s], 0.0)
        acc = None
        for k in range(9):
            q = tap_shift(jnp.dot(w3st_ref[s, k], h,
                                  preferred_element_type=jnp.float32), k)
            acc = q if acc is None else acc + q
        s_raw, t_val = acc[:ch], acc[ch:]

        s_val = sc_ref[s] * jnp.tanh(s_raw + b3s_ref[s])
        t_val = t_val + b3t_ref[s]
        xa = (x[:ch] + t_val) * jnp.exp(s_val)
        x = jnp.concatenate([xa, x[ch:]], axis=0)
        ldj_rows = ldj_rows + s_val

    xo_ref[0] = x

    ldj_lanes = jnp.sum(ldj_rows, axis=0, keepdims=True)
    if g == 1:
        delta = jnp.sum(ldj_lanes, axis=1, keepdims=True)
    else:
        row = jax.lax.broadcasted_iota(jnp.int32, (g, L), 0)
        seg = row == lane // HW
        delta = jnp.sum(jnp.where(seg, ldj_lanes, 0.0), axis=1,
                        keepdims=True)
    ldj_ref[0] = jnp.broadcast_to(delta, (g, 128))


def _const_spec(a):
    nd = a.ndim
    return pl.BlockSpec(a.shape, lambda b, _n=nd: (0,) * _n)


def _pick_group(N, HW):
    for g in range(1, N + 1):
        if N % g == 0 and (g * HW) % 128 == 0:
            return g
    return N


def _run_level(x, H, W, p):
    N, C, HW = x.shape
    ch = C // 2
    g = _pick_group(N, HW)
    B = N // g
    L = g * HW
    if g == 1:
        xb = x
    else:
        xb = x.reshape(B, g, C, HW).transpose(0, 2, 1, 3).reshape(B, C, L)

    body = functools.partial(_level_body, H=H, W=W, g=g, ch=ch)
    xo, delta = pl.pallas_call(
        body,
        grid=(B,),
        out_shape=(
            jax.ShapeDtypeStruct((B, C, L), jnp.float32),
            jax.ShapeDtypeStruct((B, g, 128), jnp.float32),
        ),
        in_specs=[pl.BlockSpec((1, C, L), lambda b: (b, 0, 0))]
        + [_const_spec(p[k]) for k in
           ("M", "bv", "w1", "b1", "w2", "b2", "w3st",
            "b3s", "b3t", "sc")],
        out_specs=(
            pl.BlockSpec((1, C, L), lambda b: (b, 0, 0)),
            pl.BlockSpec((1, g, 128), lambda b: (b, 0, 0)),
        ),
        compiler_params=pltpu.CompilerParams(
            dimension_semantics=("parallel",)),
    )(xb, p["M"], p["bv"], p["w1"], p["b1"], p["w2"], p["b2"],
      p["w3st"], p["b3s"], p["b3t"], p["sc"])

    if g != 1:
        xo = xo.reshape(B, C, g, HW).transpose(0, 2, 1, 3).reshape(N, C, HW)
    return xo, delta.reshape(N, 128)[:, 0]


def _prep_level(M, bvec, w1, b1, w2, b2, w3s, w3t, b3s, b3t, scale):
    C = M.shape[1]
    ch = C // 2
    w3st = jnp.concatenate([w3s[:, :, :ch], w3t[:, :, :ch]], axis=2)
    return {"M": M, "bv": bvec, "w1": w1, "b1": b1, "w2": w2, "b2": b2,
            "w3st": w3st, "b3s": b3s[:, :ch], "b3t": b3t[:, :ch],
            "sc": scale[:, :ch]}


def _squeeze(x, H, W):
    N, C, _ = x.shape
    x = x.reshape(N, C, H // 2, 2, W // 2, 2)
    x = x.transpose(0, 1, 3, 5, 2, 4)
    return x.reshape(N, 4 * C, (H * W) // 4)


def _unsqueeze(x, H2, W2):
    N, C4, _ = x.shape
    x = x.reshape(N, C4 // 4, 2, 2, H2, W2)
    x = x.transpose(0, 1, 4, 2, 5, 3)
    return x.reshape(N, C4 // 4, H2 * 2 * W2 * 2)


def _forward(x, sldj, levels, i, H, W):
    prep, ldj_const = levels[i]
    x, delta = _run_level(x, H, W, prep)
    sldj = sldj + ldj_const + delta
    if i + 1 < len(levels):
        x = _squeeze(x, H, W)
        c4 = x.shape[1]
        x1, x2 = x[:, : c4 // 2], x[:, c4 // 2:]
        x1, sldj = _forward(x1, sldj, levels, i + 1, H // 2, W // 2)
        x = _unsqueeze(jnp.concatenate([x1, x2], axis=1), H // 2, W // 2)
    return x, sldj


def kernel(x, sldj,
           l0_M, l0_bvec, l0_w1, l0_b1, l0_w2, l0_b2, l0_w3s, l0_w3t,
           l0_b3s, l0_b3t, l0_scale, l0_ldj_const,
           l1_M, l1_bvec, l1_w1, l1_b1, l1_w2, l1_b2, l1_w3s, l1_w3t,
           l1_b3s, l1_b3t, l1_scale, l1_ldj_const,
           l2_M, l2_bvec, l2_w1, l2_b1, l2_w2, l2_b2, l2_w3s, l2_w3t,
           l2_b3s, l2_b3t, l2_scale, l2_ldj_const):
    levels = [
        (_prep_level(l0_M, l0_bvec, l0_w1, l0_b1, l0_w2, l0_b2, l0_w3s,
                     l0_w3t, l0_b3s, l0_b3t, l0_scale), l0_ldj_const),
        (_prep_level(l1_M, l1_bvec, l1_w1, l1_b1, l1_w2, l1_b2, l1_w3s,
                     l1_w3t, l1_b3s, l1_b3t, l1_scale), l1_ldj_const),
        (_prep_level(l2_M, l2_bvec, l2_w1, l2_b1, l2_w2, l2_b2, l2_w3s,
                     l2_w3t, l2_b3s, l2_b3t, l2_scale), l2_ldj_const),
    ]
    n, c, h, w = x.shape
    r = x.reshape(n, c, h * w)
    r, sldj = _forward(r, sldj, levels, 0, h, w)
    return r.reshape(n, c, h, w), sldj


# speedup vs baseline: 1.3363x; 1.0188x over previous
import functools

import jax
import jax.numpy as jnp
from jax.experimental import pallas as pl
from jax.experimental.pallas import tpu as pltpu

_TAPS = tuple((oy, ox) for oy in (-1, 0, 1) for ox in (-1, 0, 1))


def _level_body(x_ref, m_ref, bv_ref, w1_ref, b1_ref, w2_ref, b2_ref,
                w3st_ref, b3s_ref, b3t_ref, sc_ref,
                xo_ref, ldj_ref, *, H, W, P, ch):
    L = H * W
    num_steps = m_ref.shape[0]

    lane = jax.lax.broadcasted_iota(jnp.int32, (1, L), 1)
    xpos = lane % W
    ypos = lane // W

    masks = []
    for oy, ox in _TAPS:
        m = None
        if oy == -1:
            m = ypos >= 1
        elif oy == 1:
            m = ypos <= H - 2
        if ox == -1:
            mx = xpos >= 1
            m = mx if m is None else jnp.logical_and(m, mx)
        elif ox == 1:
            mx = xpos <= W - 2
            m = mx if m is None else jnp.logical_and(m, mx)
        masks.append(m)

    def tap_shift(a, k):
        oy, ox = _TAPS[k]
        off = oy * W + ox
        out = a if off == 0 else pltpu.roll(a, (-off) % L, axis=1)
        return out if masks[k] is None else jnp.where(masks[k], out, 0.0)

    def stage1(x, s):
        x = jnp.dot(m_ref[s], x, preferred_element_type=jnp.float32) + bv_ref[s]
        pre = None
        for k in range(9):
            t = jnp.dot(w1_ref[s, k], tap_shift(x, k),
                        preferred_element_type=jnp.float32)
            pre = t if pre is None else pre + t
        return x, jnp.maximum(pre + b1_ref[s], 0.0)

    def stage2(x, h, s, ldj):
        h = jnp.maximum(jnp.dot(w2_ref[s], h, preferred_element_type=jnp.float32)
                        + b2_ref[s], 0.0)
        acc = None
        for k in range(9):
            q = tap_shift(jnp.dot(w3st_ref[s, k], h,
                                  preferred_element_type=jnp.float32), k)
            acc = q if acc is None else acc + q
        s_val = sc_ref[s] * jnp.tanh(acc[:ch] + b3s_ref[s])
        t_val = acc[ch:] + b3t_ref[s]
        xa = (x[:ch] + t_val) * jnp.exp(s_val)
        return jnp.concatenate([xa, x[ch:]], axis=0), ldj + s_val

    if P == 1:
        x = x_ref[0, 0]
        ldj = jnp.zeros((ch, L), jnp.float32)
        for s in range(num_steps):
            x, h = stage1(x, s)
            x, ldj = stage2(x, h, s, ldj)
        finals = [(x, ldj)]
    else:
        xa_, xb_ = x_ref[0, 0], x_ref[0, 1]
        la, lb = (jnp.zeros((ch, L), jnp.float32),) * 2
        xa_, ha = stage1(xa_, 0)
        for s in range(num_steps):
            xb_, hb = stage1(xb_, s)
            xa_, la = stage2(xa_, ha, s, la)
            if s + 1 < num_steps:
                xa_, ha = stage1(xa_, s + 1)
            xb_, lb = stage2(xb_, hb, s, lb)
        finals = [(xa_, la), (xb_, lb)]

    deltas = []
    for p, (x, ldj) in enumerate(finals):
        xo_ref[0, p] = x
        delta = jnp.sum(jnp.sum(ldj, axis=0, keepdims=True),
                        axis=1, keepdims=True)
        deltas.append(jnp.broadcast_to(delta, (1, 128)))
    ldj_ref[0] = (deltas[0] if len(deltas) == 1
                  else jnp.concatenate(deltas, axis=0))


def _const_spec(a):
    nd = a.ndim
    return pl.BlockSpec(a.shape, lambda b, _n=nd: (0,) * _n)


def _run_level(x, H, W, p):
    N, C, HW = x.shape
    ch = C // 2
    P = 2 if N % 2 == 0 and N >= 4 else 1
    B = N // P
    xb = x.reshape(B, P, C, HW)

    body = functools.partial(_level_body, H=H, W=W, P=P, ch=ch)
    xo, delta = pl.pallas_call(
        body,
        grid=(B,),
        out_shape=(
            jax.ShapeDtypeStruct((B, P, C, HW), jnp.float32),
            jax.ShapeDtypeStruct((B, P, 128), jnp.float32),
        ),
        in_specs=[pl.BlockSpec((1, P, C, HW), lambda b: (b, 0, 0, 0))]
        + [_const_spec(p[k]) for k in
           ("M", "bv", "w1", "b1", "w2", "b2", "w3st",
            "b3s", "b3t", "sc")],
        out_specs=(
            pl.BlockSpec((1, P, C, HW), lambda b: (b, 0, 0, 0)),
            pl.BlockSpec((1, P, 128), lambda b: (b, 0, 0)),
        ),
        compiler_params=pltpu.CompilerParams(
            dimension_semantics=("parallel",)),
    )(xb, p["M"], p["bv"], p["w1"], p["b1"], p["w2"], p["b2"],
      p["w3st"], p["b3s"], p["b3t"], p["sc"])

    return xo.reshape(N, C, HW), delta.reshape(N, 128)[:, 0]


def _prep_level(M, bvec, w1, b1, w2, b2, w3s, w3t, b3s, b3t, scale):
    C = M.shape[1]
    ch = C // 2
    w3st = jnp.concatenate([w3s[:, :, :ch], w3t[:, :, :ch]], axis=2)
    return {"M": M, "bv": bvec, "w1": w1, "b1": b1, "w2": w2, "b2": b2,
            "w3st": w3st, "b3s": b3s[:, :ch], "b3t": b3t[:, :ch],
            "sc": scale[:, :ch]}


def _squeeze(x, H, W):
    N, C, _ = x.shape
    x = x.reshape(N, C, H // 2, 2, W // 2, 2)
    x = x.transpose(0, 1, 3, 5, 2, 4)
    return x.reshape(N, 4 * C, (H * W) // 4)


def _unsqueeze(x, H2, W2):
    N, C4, _ = x.shape
    x = x.reshape(N, C4 // 4, 2, 2, H2, W2)
    x = x.transpose(0, 1, 4, 2, 5, 3)
    return x.reshape(N, C4 // 4, H2 * 2 * W2 * 2)


def _forward(x, sldj, levels, i, H, W):
    prep, ldj_const = levels[i]
    x, delta = _run_level(x, H, W, prep)
    sldj = sldj + ldj_const + delta
    if i + 1 < len(levels):
        x = _squeeze(x, H, W)
        c4 = x.shape[1]
        x1, x2 = x[:, : c4 // 2], x[:, c4 // 2:]
        x1, sldj = _forward(x1, sldj, levels, i + 1, H // 2, W // 2)
        x = _unsqueeze(jnp.concatenate([x1, x2], axis=1), H // 2, W // 2)
    return x, sldj


def kernel(x, sldj,
           l0_M, l0_bvec, l0_w1, l0_b1, l0_w2, l0_b2, l0_w3s, l0_w3t,
           l0_b3s, l0_b3t, l0_scale, l0_ldj_const,
           l1_M, l1_bvec, l1_w1, l1_b1, l1_w2, l1_b2, l1_w3s, l1_w3t,
           l1_b3s, l1_b3t, l1_scale, l1_ldj_const,
           l2_M, l2_bvec, l2_w1, l2_b1, l2_w2, l2_b2, l2_w3s, l2_w3t,
           l2_b3s, l2_b3t, l2_scale, l2_ldj_const):
    levels = [
        (_prep_level(l0_M, l0_bvec, l0_w1, l0_b1, l0_w2, l0_b2, l0_w3s,
                     l0_w3t, l0_b3s, l0_b3t, l0_scale), l0_ldj_const),
        (_prep_level(l1_M, l1_bvec, l1_w1, l1_b1, l1_w2, l1_b2, l1_w3s,
                     l1_w3t, l1_b3s, l1_b3t, l1_scale), l1_ldj_const),
        (_prep_level(l2_M, l2_bvec, l2_w1, l2_b1, l2_w2, l2_b2, l2_w3s,
                     l2_w3t, l2_b3s, l2_b3t, l2_scale), l2_ldj_const),
    ]
    n, c, h, w = x.shape
    r = x.reshape(n, c, h * w)
    r, sldj = _forward(r, sldj, levels, 0, h, w)
    return r.reshape(n, c, h, w), sldj
